# baseline (speedup 1.0000x reference)
import functools

import jax
import jax.numpy as jnp
from jax import lax
from jax.experimental import pallas as pl
from jax.experimental.pallas import tpu as pltpu
from jax.experimental.pallas import tpu_sc as plsc

N = 1048576
G = 1024
DT = 1e-06
CV = 5e-14
VMIN, VMAX = 0.0, 1.0
UMIN, UMAX = 0.0, 1.0
J_PER_X = (G - 1) / (VMAX - VMIN)
I_PER_Y = (G - 1) / (UMAX - UMIN)

NC = 2
NS = 16
NW = NC * NS
PER_W = N // NW
C = 8192
NCHUNK = PER_W // C
L = 16


def _sc_body(in_hbm, v_hbm, u_hbm, icv_hbm, axon_hbm,
             spk_hbm, vnew_hbm,
             v_buf, u_buf, in_buf, idx_buf, iv_buf, ax_buf, sem):
    wid = lax.axis_index("s") * NC + lax.axis_index("c")
    base = wid * PER_W

    def chunk_body(t, carry):
        off = base + t * C
        pltpu.sync_copy(v_hbm.at[pl.ds(off, C)], v_buf)
        pltpu.sync_copy(u_hbm.at[pl.ds(off, C)], u_buf)
        pltpu.sync_copy(in_hbm.at[pl.ds(off, C)], in_buf)

        def idx_body(i, c):
            s = i * L
            u16 = u_buf[pl.ds(s, L)]
            v16 = v_buf[pl.ds(s, L)]
            ii = jnp.clip((u16 * I_PER_Y).astype(jnp.int32), 0, G - 1)
            jj = jnp.clip((v16 * J_PER_X).astype(jnp.int32), 0, G - 1)
            idx_buf[pl.ds(s, L)] = ii * G + jj
            return c

        lax.fori_loop(0, C // L, idx_body, 0)

        cp1 = pltpu.async_copy(icv_hbm.at[idx_buf], iv_buf, sem)
        cp2 = pltpu.async_copy(axon_hbm.at[idx_buf], ax_buf, sem)
        cp1.wait()
        cp2.wait()

        def out_body(i, c):
            s = i * L
            iv = iv_buf[pl.ds(s, L)]
            vv = v_buf[pl.ds(s, L)]
            xx = in_buf[pl.ds(s, L)]
            vn = vv + (iv + xx) / CV * DT
            v_buf[pl.ds(s, L)] = jnp.clip(vn, VMIN, VMAX)
            return c

        lax.fori_loop(0, C // L, out_body, 0)

        pltpu.sync_copy(ax_buf, spk_hbm.at[pl.ds(off, C)])
        pltpu.sync_copy(v_buf, vnew_hbm.at[pl.ds(off, C)])
        return carry

    lax.fori_loop(0, NCHUNK, chunk_body, 0)


@jax.jit
def _run(inp, v, u, icv_flat, axon_flat):
    f32 = jnp.float32
    k = pl.kernel(
        _sc_body,
        out_type=(
            jax.ShapeDtypeStruct((N,), f32),
            jax.ShapeDtypeStruct((N,), f32),
        ),
        mesh=plsc.VectorSubcoreMesh(core_axis_name="c", subcore_axis_name="s"),
        scratch_types=[
            pltpu.VMEM((C,), f32),
            pltpu.VMEM((C,), f32),
            pltpu.VMEM((C,), f32),
            pltpu.VMEM((C,), jnp.int32),
            pltpu.VMEM((C,), f32),
            pltpu.VMEM((C,), f32),
            pltpu.SemaphoreType.DMA,
        ],
    )
    return k(inp, v, u, icv_flat, axon_flat)


def kernel(input, v, u, iCv, iCu, axon, num_steps):
    del iCu, num_steps
    spikes, v_new = _run(input, v, u, iCv.reshape(-1), axon.reshape(-1))
    return (spikes, v_new)


# speedup vs baseline: 1.7563x; 1.7563x over previous
import jax
import jax.numpy as jnp
from jax import lax
from jax.experimental import pallas as pl
from jax.experimental.pallas import tpu as pltpu
from jax.experimental.pallas import tpu_sc as plsc

N = 1048576
G = 1024
DT = 1e-06
CV = 5e-14
VMIN, VMAX = 0.0, 1.0
UMIN, UMAX = 0.0, 1.0
J_PER_X = (G - 1) / (VMAX - VMIN)
I_PER_Y = (G - 1) / (UMAX - UMIN)

NC = 2
NS = 16
NW = NC * NS
PER_W = N // NW
C = 8192
NCHUNK = PER_W // C
L = 16


def _pack_body(icv_ref, ax_ref, out_ref):
    icv_i = lax.bitcast_convert_type(icv_ref[...], jnp.int32)
    bit = (ax_ref[...] != 0.0).astype(jnp.int32)
    out_ref[...] = lax.bitcast_convert_type((icv_i & jnp.int32(-2)) | bit,
                                            jnp.float32)


def _sc_body(in_hbm, v_hbm, u_hbm, tab_hbm, spk_hbm, vnew_hbm,
             v_bufs, u_bufs, in_bufs, idx0, idx1, val0, val1, sem0, sem1):
    wid = lax.axis_index("s") * NC + lax.axis_index("c")
    base = wid * PER_W
    sems = (sem0, sem1)
    idxs = (idx0, idx1)
    vals = (val0, val1)

    def idx_stage(b):
        u_ref = u_bufs.at[b]
        v_ref = v_bufs.at[b]
        idx_ref = idxs[b]

        @plsc.parallel_loop(0, C // L, unroll=4)
        def _(i):
            s = i * L
            ii = jnp.clip((u_ref[pl.ds(s, L)] * I_PER_Y).astype(jnp.int32),
                          0, G - 1)
            jj = jnp.clip((v_ref[pl.ds(s, L)] * J_PER_X).astype(jnp.int32),
                          0, G - 1)
            idx_ref[pl.ds(s, L)] = ii * G + jj

    def out_stage(b):
        v_ref = v_bufs.at[b]
        u_ref = u_bufs.at[b]
        in_ref = in_bufs.at[b]
        val_ref = vals[b]

        @plsc.parallel_loop(0, C // L, unroll=4)
        def _(i):
            s = i * L
            val_i = lax.bitcast_convert_type(val_ref[pl.ds(s, L)], jnp.int32)
            spike = (val_i & 1).astype(jnp.float32)
            iv = lax.bitcast_convert_type(val_i & jnp.int32(-2), jnp.float32)
            vn = v_ref[pl.ds(s, L)] + (iv + in_ref[pl.ds(s, L)]) / CV * DT
            v_ref[pl.ds(s, L)] = jnp.clip(vn, VMIN, VMAX)
            u_ref[pl.ds(s, L)] = spike

    cps = [None] * NCHUNK
    for t in range(NCHUNK):
        b = t % 2
        off = base + t * C
        pltpu.sync_copy(v_hbm.at[pl.ds(off, C)], v_bufs.at[b])
        pltpu.sync_copy(u_hbm.at[pl.ds(off, C)], u_bufs.at[b])
        idx_stage(b)
        cps[t] = pltpu.async_copy(tab_hbm.at[idxs[b]],
                                  vals[b], sems[b])
        pltpu.sync_copy(in_hbm.at[pl.ds(off, C)], in_bufs.at[b])
        if t > 0:
            pb = (t - 1) % 2
            poff = base + (t - 1) * C
            cps[t - 1].wait()
            out_stage(pb)
            pltpu.sync_copy(v_bufs.at[pb], vnew_hbm.at[pl.ds(poff, C)])
            pltpu.sync_copy(u_bufs.at[pb], spk_hbm.at[pl.ds(poff, C)])
    lb = (NCHUNK - 1) % 2
    loff = base + (NCHUNK - 1) * C
    cps[NCHUNK - 1].wait()
    out_stage(lb)
    pltpu.sync_copy(v_bufs.at[lb], vnew_hbm.at[pl.ds(loff, C)])
    pltpu.sync_copy(u_bufs.at[lb], spk_hbm.at[pl.ds(loff, C)])


@jax.jit
def _run(inp, v, u, icv, axon):
    f32 = jnp.float32
    packed = pl.pallas_call(
        _pack_body,
        out_shape=jax.ShapeDtypeStruct((G, G), f32),
    )(icv, axon)

    k = pl.kernel(
        _sc_body,
        out_type=(
            jax.ShapeDtypeStruct((N,), f32),
            jax.ShapeDtypeStruct((N,), f32),
        ),
        mesh=plsc.VectorSubcoreMesh(core_axis_name="c", subcore_axis_name="s"),
        scratch_types=[
            pltpu.VMEM((2, C), f32),
            pltpu.VMEM((2, C), f32),
            pltpu.VMEM((2, C), f32),
            pltpu.VMEM((C,), jnp.int32),
            pltpu.VMEM((C,), jnp.int32),
            pltpu.VMEM((C,), f32),
            pltpu.VMEM((C,), f32),
            pltpu.SemaphoreType.DMA,
            pltpu.SemaphoreType.DMA,
        ],
    )
    return k(inp, v, u, packed.reshape(-1))


def kernel(input, v, u, iCv, iCu, axon, num_steps):
    del iCu, num_steps
    spikes, v_new = _run(input, v, u, iCv, axon)
    return (spikes, v_new)
